# Initial kernel scaffold; baseline (speedup 1.0000x reference)
import functools

import jax
import jax.numpy as jnp
from jax import lax
from jax.experimental import pallas as pl
from jax.experimental.pallas import tpu as pltpu
from jax.experimental.pallas import tpu_sc as plsc

N_USERS = 25000
N_ITEMS = 25000
N = N_USERS + N_ITEMS
DH = 32
N_LAYERS = 3
E = 800000
SUB = 128
SUBS_PER_TILE = 400
NTILES = 16
E_PAD = NTILES * SUBS_PER_TILE * SUB
NROWS = E_PAD // SUB
BLK = 16
NBLK = SUBS_PER_TILE // BLK
RPT = N // NTILES
ZCH = 125
NCH = RPT // ZCH


def _sc_propagate(x0h, col2d, row2d, val2d):
    mesh = plsc.VectorSubcoreMesh(core_axis_name="c", subcore_axis_name="s")

    @functools.partial(
        pl.kernel,
        out_type=[
            jax.ShapeDtypeStruct((2, 2, N, DH), jnp.float32),
            jax.ShapeDtypeStruct((2, N, DH), jnp.float32),
        ],
        mesh=mesh,
        scratch_types=[
            pltpu.VMEM_SHARED((N, DH), jnp.float32),
            pltpu.VMEM((BLK, SUB), jnp.int32),
            pltpu.VMEM((BLK, SUB), jnp.int32),
            pltpu.VMEM((BLK, SUB), jnp.float32),
            pltpu.VMEM((BLK, SUB, DH), jnp.float32),
            pltpu.VMEM((ZCH, DH), jnp.float32),
            pltpu.VMEM((ZCH, DH), jnp.float32),
            pltpu.VMEM((ZCH, DH), jnp.float32),
            pltpu.VMEM((ZCH, DH), jnp.float32),
            pltpu.VMEM((ZCH, DH), jnp.float32),
            pltpu.SemaphoreType.DMA,
            pltpu.SemaphoreType.DMA,
            pltpu.SemaphoreType.DMA,
        ],
    )
    def k(x0h_hbm, col_hbm, row_hbm, val_hbm, xs_hbm, fin_hbm,
          accum, colbuf, rowbuf, valbuf, gbuf, zbuf, tbuf, t0, t1, t2,
          gsem, ssem, dsem):
        c = lax.axis_index("c")
        s = lax.axis_index("s")
        ebase = s * SUBS_PER_TILE
        abase = s * RPT

        zero16 = jnp.zeros((16,), jnp.float32)

        @pl.loop(0, ZCH)
        def _(i):
            zbuf[i, pl.ds(0, 16)] = zero16
            zbuf[i, pl.ds(16, 16)] = zero16

        def zero_accum():
            @pl.loop(0, NCH)
            def _(i):
                pltpu.sync_copy(zbuf, accum.at[pl.ds(abase + i * ZCH, ZCH)])

        def edge_pass(src_hbm):
            @pl.loop(0, NBLK)
            def _(blk):
                r0 = ebase + blk * BLK
                pltpu.sync_copy(col_hbm.at[pl.ds(r0, BLK)], colbuf)
                pltpu.sync_copy(row_hbm.at[pl.ds(r0, BLK)], rowbuf)
                pltpu.sync_copy(val_hbm.at[pl.ds(r0, BLK)], valbuf)
                gds = [
                    pltpu.async_copy(src_hbm.at[colbuf.at[r]], gbuf.at[r], gsem)
                    for r in range(BLK)
                ]
                for d in gds:
                    d.wait()
                sds = []
                for r in range(BLK):
                    @pl.loop(0, SUB, unroll=8)
                    def _(kk, r=r):
                        v = valbuf[r, kk]
                        gbuf[r, kk, pl.ds(0, 16)] = gbuf[r, kk, pl.ds(0, 16)] * v
                        gbuf[r, kk, pl.ds(16, 16)] = gbuf[r, kk, pl.ds(16, 16)] * v
                    sds.append(pltpu.async_copy(
                        gbuf.at[r], accum.at[rowbuf.at[r]], ssem, add=True))
                for d in sds:
                    d.wait()

        def drain_plain(layer):
            @pl.loop(0, NCH)
            def _(i):
                rows = pl.ds(abase + i * ZCH, ZCH)
                pltpu.sync_copy(accum.at[rows], tbuf)
                pltpu.sync_copy(tbuf, xs_hbm.at[layer, c, rows])

        def drain_mean():
            @pl.loop(0, NCH)
            def _(i):
                rows = pl.ds(abase + i * ZCH, ZCH)
                pltpu.sync_copy(accum.at[rows], tbuf)
                d0 = pltpu.async_copy(x0h_hbm.at[c, rows], t0, dsem)
                d1 = pltpu.async_copy(xs_hbm.at[0, c, rows], t1, dsem)
                d2 = pltpu.async_copy(xs_hbm.at[1, c, rows], t2, dsem)
                d0.wait()
                d1.wait()
                d2.wait()

                @pl.loop(0, ZCH, unroll=4)
                def _(j):
                    for h in (0, 16):
                        hs = pl.ds(h, 16)
                        tbuf[j, hs] = (tbuf[j, hs] + t0[j, hs]
                                       + t1[j, hs] + t2[j, hs]) * 0.25
                pltpu.sync_copy(tbuf, fin_hbm.at[c, rows])

        for layer in range(N_LAYERS):
            zero_accum()
            plsc.subcore_barrier()
            if layer == 0:
                edge_pass(x0h_hbm.at[c])
            else:
                edge_pass(xs_hbm.at[layer - 1, c])
            plsc.subcore_barrier()
            if layer < 2:
                drain_plain(layer)
            else:
                drain_mean()
            plsc.subcore_barrier()

    return k(x0h, col2d, row2d, val2d)


def kernel(edge_index, edge_values, user_emb, item_emb):
    all_emb = jnp.concatenate([user_emb, item_emb], axis=0)
    x0h = jnp.stack([all_emb[:, :DH], all_emb[:, DH:]], axis=0)
    pad = E_PAD - E
    col = jnp.concatenate([edge_index[1], jnp.zeros((pad,), jnp.int32)])
    row = jnp.concatenate([edge_index[0], jnp.zeros((pad,), jnp.int32)])
    val = jnp.concatenate([edge_values, jnp.zeros((pad,), jnp.float32)])
    xs, finalh = _sc_propagate(
        x0h, col.reshape(NROWS, SUB), row.reshape(NROWS, SUB),
        val.reshape(NROWS, SUB))
    del xs
    final = jnp.concatenate([finalh[0], finalh[1]], axis=1)
    return final[:N_USERS], final[N_USERS:]


# speedup vs baseline: 5.0587x; 5.0587x over previous
import functools

import jax
import jax.numpy as jnp
from jax import lax
from jax.experimental import pallas as pl
from jax.experimental.pallas import tpu as pltpu
from jax.experimental.pallas import tpu_sc as plsc

N_USERS = 25000
N_ITEMS = 25000
N = N_USERS + N_ITEMS
DH = 32
N_LAYERS = 3
E = 800000
SUB = 128
SUBS_PER_TILE = 400
NTILES = 16
E_PAD = NTILES * SUBS_PER_TILE * SUB
NROWS = E_PAD // SUB
BLK = 4
NBLK = SUBS_PER_TILE // BLK
RPT = N // NTILES
ZCH = 25
NCH = RPT // ZCH


def _sc_propagate(x0h, col2d, row2d, val2d):
    mesh = plsc.VectorSubcoreMesh(core_axis_name="c", subcore_axis_name="s")

    @functools.partial(
        pl.kernel,
        out_type=[
            jax.ShapeDtypeStruct((2, 2, N, DH), jnp.float32),
            jax.ShapeDtypeStruct((2, N, DH), jnp.float32),
        ],
        mesh=mesh,
        compiler_params=pltpu.CompilerParams(use_tc_tiling_on_sc=False),
        scratch_types=[
            pltpu.VMEM_SHARED((N, DH), jnp.float32),
            pltpu.VMEM((BLK, SUB), jnp.int32),
            pltpu.VMEM((BLK, SUB), jnp.int32),
            pltpu.VMEM((BLK, SUB), jnp.float32),
            pltpu.VMEM((BLK, SUB, DH), jnp.float32),
            pltpu.VMEM((ZCH, DH), jnp.float32),
            pltpu.VMEM((ZCH, DH), jnp.float32),
            pltpu.VMEM((ZCH, DH), jnp.float32),
            pltpu.VMEM((ZCH, DH), jnp.float32),
            pltpu.VMEM((ZCH, DH), jnp.float32),
            pltpu.SemaphoreType.DMA,
            pltpu.SemaphoreType.DMA,
            pltpu.SemaphoreType.DMA,
        ],
    )
    def k(x0h_hbm, col_hbm, row_hbm, val_hbm, xs_hbm, fin_hbm,
          accum, colbuf, rowbuf, valbuf, gbuf, zbuf, tbuf, t0, t1, t2,
          gsem, ssem, dsem):
        c = lax.axis_index("c")
        s = lax.axis_index("s")
        ebase = s * SUBS_PER_TILE
        abase = s * RPT

        zero16 = jnp.zeros((16,), jnp.float32)

        @pl.loop(0, ZCH)
        def _(i):
            zbuf[i, pl.ds(0, 16)] = zero16
            zbuf[i, pl.ds(16, 16)] = zero16

        def zero_accum():
            @pl.loop(0, NCH)
            def _(i):
                pltpu.sync_copy(zbuf, accum.at[pl.ds(abase + i * ZCH, ZCH)])

        def edge_pass(src_hbm):
            @pl.loop(0, NBLK)
            def _(blk):
                r0 = ebase + blk * BLK
                pltpu.sync_copy(col_hbm.at[pl.ds(r0, BLK)], colbuf)
                pltpu.sync_copy(row_hbm.at[pl.ds(r0, BLK)], rowbuf)
                pltpu.sync_copy(val_hbm.at[pl.ds(r0, BLK)], valbuf)
                gds = [
                    pltpu.async_copy(src_hbm.at[colbuf.at[r]], gbuf.at[r], gsem)
                    for r in range(BLK)
                ]
                for d in gds:
                    d.wait()
                sds = []
                for r in range(BLK):
                    @pl.loop(0, SUB // 16)
                    def _(g, r=r):
                        v16 = valbuf[r, pl.ds(g * 16, 16)]
                        for j in range(16):
                            kk = g * 16 + j
                            v = v16[j]
                            gbuf[r, kk, pl.ds(0, 16)] = gbuf[r, kk, pl.ds(0, 16)] * v
                            gbuf[r, kk, pl.ds(16, 16)] = gbuf[r, kk, pl.ds(16, 16)] * v
                    sds.append(pltpu.async_copy(
                        gbuf.at[r], accum.at[rowbuf.at[r]], ssem, add=True))
                for d in sds:
                    d.wait()

        def drain_plain(layer):
            @pl.loop(0, NCH)
            def _(i):
                rows = pl.ds(abase + i * ZCH, ZCH)
                pltpu.sync_copy(accum.at[rows], tbuf)
                pltpu.sync_copy(tbuf, xs_hbm.at[layer, c, rows])

        def drain_mean():
            @pl.loop(0, NCH)
            def _(i):
                rows = pl.ds(abase + i * ZCH, ZCH)
                pltpu.sync_copy(accum.at[rows], tbuf)
                d0 = pltpu.async_copy(x0h_hbm.at[c, rows], t0, dsem)
                d1 = pltpu.async_copy(xs_hbm.at[0, c, rows], t1, dsem)
                d2 = pltpu.async_copy(xs_hbm.at[1, c, rows], t2, dsem)
                d0.wait()
                d1.wait()
                d2.wait()

                @pl.loop(0, ZCH, unroll=4)
                def _(j):
                    for h in (0, 16):
                        hs = pl.ds(h, 16)
                        tbuf[j, hs] = (tbuf[j, hs] + t0[j, hs]
                                       + t1[j, hs] + t2[j, hs]) * 0.25
                pltpu.sync_copy(tbuf, fin_hbm.at[c, rows])

        for layer in range(N_LAYERS):
            zero_accum()
            plsc.subcore_barrier()
            if layer == 0:
                edge_pass(x0h_hbm.at[c])
            else:
                edge_pass(xs_hbm.at[layer - 1, c])
            plsc.subcore_barrier()
            if layer < 2:
                drain_plain(layer)
            else:
                drain_mean()
            plsc.subcore_barrier()

    return k(x0h, col2d, row2d, val2d)


def kernel(edge_index, edge_values, user_emb, item_emb):
    all_emb = jnp.concatenate([user_emb, item_emb], axis=0)
    x0h = jnp.stack([all_emb[:, :DH], all_emb[:, DH:]], axis=0)
    pad = E_PAD - E
    col = jnp.concatenate([edge_index[1], jnp.zeros((pad,), jnp.int32)])
    row = jnp.concatenate([edge_index[0], jnp.zeros((pad,), jnp.int32)])
    val = jnp.concatenate([edge_values, jnp.zeros((pad,), jnp.float32)])
    xs, finalh = _sc_propagate(
        x0h, col.reshape(NROWS, SUB), row.reshape(NROWS, SUB),
        val.reshape(NROWS, SUB))
    del xs
    final = jnp.concatenate([finalh[0], finalh[1]], axis=1)
    return final[:N_USERS], final[N_USERS:]
